# baseline (speedup 1.0000x reference)
import functools

import jax
import jax.numpy as jnp
from jax.experimental import pallas as pl
from jax.experimental.pallas import tpu as pltpu

_NU = 0.001
_ALPHA = 0.1
_K = 3
_J = 3
_SCALE = 32
_NSEL = _K + _J
_LANES = 128

_SORT8 = [(0, 1), (2, 3), (4, 5), (6, 7),
          (0, 2), (1, 3), (4, 6), (5, 7),
          (1, 2), (5, 6),
          (0, 4), (1, 5), (2, 6), (3, 7),
          (2, 4), (3, 5),
          (1, 2), (3, 4), (5, 6)]


def _dsvdd_body(r_ref, phi_ref, c_ref, loss_ref, score_ref, colsqh_ref, cbf_ref,
                *, n_steps, blk_n, m):
    i = pl.program_id(0)
    n_chunks = m // _LANES

    @pl.when(i == 0)
    def _init():
        c = c_ref[...]
        colsqh_ref[...] = 0.5 * jnp.sum(c * c, axis=0, keepdims=True)
        cbf_ref[...] = c.astype(jnp.bfloat16)
        loss_ref[...] = jnp.zeros_like(loss_ref)

    xb = phi_ref[...].astype(jnp.bfloat16)
    ones = jnp.ones((phi_ref.shape[1], 1), dtype=jnp.bfloat16)
    rowsq = jnp.dot(xb * xb, ones, preferred_element_type=jnp.float32)

    n_sub = 4
    rs = blk_n // n_sub
    colsqh = colsqh_ref[...]
    cbf = cbf_ref[...]
    r2 = r_ref[0, 0] * r_ref[0, 0]
    zero = jnp.float32(0.0)
    ids = jax.lax.broadcasted_iota(jnp.int32, (rs, _LANES), 1).astype(jnp.float32)
    part = []
    for sub in range(n_sub):
        lo_r, hi_r = sub * rs, (sub + 1) * rs
        acc = jnp.dot(xb[lo_r:hi_r, :], cbf, preferred_element_type=jnp.float32)

        v = [colsqh[:, c * _LANES:(c + 1) * _LANES]
             - acc[:, c * _LANES:(c + 1) * _LANES] for c in range(n_chunks)]
        for a, b in _SORT8:
            lo = jnp.minimum(v[a], v[b])
            hi = jnp.maximum(v[a], v[b])
            v[a], v[b] = lo, hi

        front = v[0]
        s = v[1:_NSEL]
        mins = []
        for k in range(_NSEL):
            mval = jnp.min(front, axis=1, keepdims=True)
            mins.append(mval)
            if k < _NSEL - 1:
                idx = jnp.min(jnp.where(front == mval, ids, jnp.float32(_LANES)),
                              axis=1, keepdims=True)
                lanemask = ids == idx
                front = jnp.where(lanemask, s[0], front)
                for j in range(_NSEL - 2 - k):
                    s[j] = jnp.where(lanemask, s[j + 1], s[j])
        dmat = (2.0 * jnp.concatenate(mins, axis=1)
                + rowsq[lo_r:hi_r, :]).T

        dk = jnp.sqrt(dmat[0:3, :])
        s0 = dk[0:1, :]
        sm0 = 1.0 / (1.0 + jnp.exp(s0 - dk[1:2, :]) + jnp.exp(s0 - dk[2:3, :]))
        score_ref[lo_r:hi_r, :] = (s0 * sm0).T

        att = jnp.maximum(dmat[0:3, :] - r2, zero)
        rep = jnp.maximum(r2 - dmat[3:6, :] - _ALPHA, zero)
        part.append(jnp.sum(att) + jnp.sum(rep))
    loss_ref[...] += sum(part)

    @pl.when(i == n_steps - 1)
    def _finalize():
        total = jnp.float32(n_steps * blk_n * _K)
        loss_ref[...] *= 1.0 / (_NU * total)


def kernel(phi_p, C, r):
    b, hw, dfeat = phi_p.shape
    m = C.shape[1]
    n = b * hw
    blk_n = 1024
    n_steps = n // blk_n

    phi = phi_p.reshape(n, dfeat)
    r_in = r.reshape(1, 1)

    body = functools.partial(_dsvdd_body, n_steps=n_steps, blk_n=blk_n, m=m)
    loss2, score2 = pl.pallas_call(
        body,
        grid=(n_steps,),
        in_specs=[
            pl.BlockSpec((1, 1), lambda i: (0, 0)),
            pl.BlockSpec((blk_n, dfeat), lambda i: (i, 0)),
            pl.BlockSpec((dfeat, m), lambda i: (0, 0)),
        ],
        out_specs=[
            pl.BlockSpec((1, 1), lambda i: (0, 0)),
            pl.BlockSpec((blk_n, 1), lambda i: (i, 0)),
        ],
        out_shape=[
            jax.ShapeDtypeStruct((1, 1), jnp.float32),
            jax.ShapeDtypeStruct((n, 1), jnp.float32),
        ],
        scratch_shapes=[
            pltpu.VMEM((1, m), jnp.float32),
            pltpu.VMEM((dfeat, m), jnp.bfloat16),
        ],
    )(r_in, phi, C)

    loss = loss2[0, 0]
    score = score2.reshape(b, _SCALE, _SCALE, 1).transpose(0, 3, 1, 2)
    return loss, score


# speedup vs baseline: 1.0372x; 1.0372x over previous
import functools

import jax
import jax.numpy as jnp
from jax.experimental import pallas as pl
from jax.experimental.pallas import tpu as pltpu

_NU = 0.001
_ALPHA = 0.1
_K = 3
_J = 3
_SCALE = 32
_NSEL = _K + _J
_LANES = 128

_SORT8 = [(0, 1), (2, 3), (4, 5), (6, 7),
          (0, 2), (1, 3), (4, 6), (5, 7),
          (1, 2), (5, 6),
          (0, 4), (1, 5), (2, 6), (3, 7),
          (2, 4), (3, 5),
          (1, 2), (3, 4), (5, 6)]


def _dsvdd_body(r_ref, phi_ref, c_ref, loss_ref, score_ref, colsqh_ref, cbf_ref,
                *, n_steps, blk_n, m):
    i = pl.program_id(0)
    n_chunks = m // _LANES

    @pl.when(i == 0)
    def _init():
        c = c_ref[...]
        colsqh_ref[...] = 0.5 * jnp.sum(c * c, axis=0, keepdims=True)
        cbf_ref[...] = c.astype(jnp.bfloat16)
        loss_ref[...] = jnp.zeros_like(loss_ref)

    xb = phi_ref[...].astype(jnp.bfloat16)
    ones = jnp.ones((phi_ref.shape[1], 1), dtype=jnp.bfloat16)
    rowsq = jnp.dot(xb * xb, ones, preferred_element_type=jnp.float32)

    n_sub = 4
    rs = blk_n // n_sub
    colsqh = colsqh_ref[...]
    cbf = cbf_ref[...]
    r2 = r_ref[0, 0] * r_ref[0, 0]
    zero = jnp.float32(0.0)
    ids = jax.lax.broadcasted_iota(jnp.int32, (rs, _LANES), 1).astype(jnp.float32)
    part = []
    accs = [jnp.dot(xb[sub * rs:(sub + 1) * rs, :], cbf,
                    preferred_element_type=jnp.float32)
            for sub in range(n_sub)]
    for sub in range(n_sub):
        lo_r, hi_r = sub * rs, (sub + 1) * rs
        acc = accs[sub]

        v = [colsqh[:, c * _LANES:(c + 1) * _LANES]
             - acc[:, c * _LANES:(c + 1) * _LANES] for c in range(n_chunks)]
        for a, b in _SORT8:
            lo = jnp.minimum(v[a], v[b])
            hi = jnp.maximum(v[a], v[b])
            v[a], v[b] = lo, hi

        front = v[0]
        s = v[1:_NSEL]
        mins = []
        for k in range(_NSEL):
            mval = jnp.min(front, axis=1, keepdims=True)
            mins.append(mval)
            if k < _NSEL - 1:
                idx = jnp.min(jnp.where(front == mval, ids, jnp.float32(_LANES)),
                              axis=1, keepdims=True)
                lanemask = ids == idx
                front = jnp.where(lanemask, s[0], front)
                for j in range(_NSEL - 2 - k):
                    s[j] = jnp.where(lanemask, s[j + 1], s[j])
        d0, d1, d2, d3, d4, d5 = [2.0 * mv + rowsq[lo_r:hi_r, :]
                                  for mv in mins]

        s0 = jnp.sqrt(d0)
        s1 = jnp.sqrt(d1)
        s2 = jnp.sqrt(d2)
        sm0 = 1.0 / (1.0 + jnp.exp(s0 - s1) + jnp.exp(s0 - s2))
        score_ref[lo_r:hi_r, :] = s0 * sm0

        att = (jnp.maximum(d0 - r2, zero) + jnp.maximum(d1 - r2, zero)
               + jnp.maximum(d2 - r2, zero))
        rep = (jnp.maximum(r2 - d3 - _ALPHA, zero)
               + jnp.maximum(r2 - d4 - _ALPHA, zero)
               + jnp.maximum(r2 - d5 - _ALPHA, zero))
        part.append(jnp.sum(att + rep))
    loss_ref[...] += sum(part)

    @pl.when(i == n_steps - 1)
    def _finalize():
        total = jnp.float32(n_steps * blk_n * _K)
        loss_ref[...] *= 1.0 / (_NU * total)


def kernel(phi_p, C, r):
    b, hw, dfeat = phi_p.shape
    m = C.shape[1]
    n = b * hw
    blk_n = 1024
    n_steps = n // blk_n

    phi = phi_p.reshape(n, dfeat)
    r_in = r.reshape(1, 1)

    body = functools.partial(_dsvdd_body, n_steps=n_steps, blk_n=blk_n, m=m)
    loss2, score2 = pl.pallas_call(
        body,
        grid=(n_steps,),
        in_specs=[
            pl.BlockSpec((1, 1), lambda i: (0, 0)),
            pl.BlockSpec((blk_n, dfeat), lambda i: (i, 0)),
            pl.BlockSpec((dfeat, m), lambda i: (0, 0)),
        ],
        out_specs=[
            pl.BlockSpec((1, 1), lambda i: (0, 0)),
            pl.BlockSpec((blk_n, 1), lambda i: (i, 0)),
        ],
        out_shape=[
            jax.ShapeDtypeStruct((1, 1), jnp.float32),
            jax.ShapeDtypeStruct((n, 1), jnp.float32),
        ],
        scratch_shapes=[
            pltpu.VMEM((1, m), jnp.float32),
            pltpu.VMEM((dfeat, m), jnp.bfloat16),
        ],
    )(r_in, phi, C)

    loss = loss2[0, 0]
    score = score2.reshape(b, _SCALE, _SCALE, 1).transpose(0, 3, 1, 2)
    return loss, score
